# baseline (speedup 1.0000x reference)
import functools

import jax
import jax.numpy as jnp
from jax import lax
from jax.experimental import pallas as pl
from jax.experimental.pallas import tpu as pltpu
from jax.experimental.pallas import tpu_sc as plsc

N = 10000
E = 160000
D = 256
H = 128
NC = 2
NS = 16
L = 16
EPT = E // NS
C = 80
MEGA = 2000
NMEGA = EPT // MEGA
MCH = MEGA // C
NBUF = 3
NCHUNK = EPT // C
ZROWS = 80
BN = 2000


def _own_range(sid):
    return sid * 640


def _sc_body(layer1, h_hbm, src_hbm, dst2_hbm, logit_hbm, b_hbm, *rest):
    if layer1:
        out_hbm, den_hbm = rest[0], rest[1]
    else:
        den_hbm, out_hbm = rest[0], rest[1]
    (acc, dacc, idx0, idx1, dst0, dst1, t0, t1,
     rows0, rows1, rows2, sem_ld0, sem_ld1, semd,
     semg0, semg1, semg2, sems0, sems1, sems2) = rest[2:]
    c = lax.axis_index("c")
    sid = lax.axis_index("s")
    rows = (rows0, rows1, rows2)
    semg = (semg0, semg1, semg2)
    sems = (sems0, sems1, sems2)
    sets = ((idx0, dst0, t0, sem_ld0), (idx1, dst1, t1, sem_ld1))

    def zero_zb(i, _):
        for j in range(H // L):
            rows0[i, pl.ds(j * L, L)] = jnp.zeros((L,), jnp.float32)
        return _
    lax.fori_loop(0, ZROWS, zero_zb, None)
    for j in range((640 // L) if layer1 else 0):
        t0[pl.ds(j * L, L)] = jnp.zeros((L,), jnp.float32)

    own = _own_range(sid)

    @pl.when(sid < NS - 1)
    def _():
        for k in range(640 // ZROWS):
            pltpu.sync_copy(rows0, acc.at[pl.ds(own + k * ZROWS, ZROWS)])
        if layer1:
            pltpu.sync_copy(t0.at[pl.ds(0, 640)], dacc.at[pl.ds(own, 640)])

    @pl.when(sid == NS - 1)
    def _():
        for k in range(400 // ZROWS):
            pltpu.sync_copy(rows0, acc.at[pl.ds(own + k * ZROWS, ZROWS)])
        if layer1:
            pltpu.sync_copy(t0.at[pl.ds(0, 400)], dacc.at[pl.ds(own, 400)])

    plsc.subcore_barrier()

    def _load_descs(s, m):
        idx_s, dst_s, t_s, sem_s = sets[s]
        base = pl.multiple_of(sid * EPT + m * MEGA, 8)
        blk = sid * NMEGA + m
        return (pltpu.make_async_copy(src_hbm.at[pl.ds(base, MEGA)], idx_s, sem_s),
                pltpu.make_async_copy(logit_hbm.at[pl.ds(base, MEGA)], t_s, sem_s),
                pltpu.make_async_copy(dst2_hbm.at[blk], dst_s, sem_s))

    def _gather_desc(s, x, b):
        return pltpu.make_async_copy(
            h_hbm.at[sets[s][0].at[pl.ds(x * C, C)]], rows[b], semg[b])

    def _scatter_desc(s, x, b):
        return pltpu.make_async_copy(rows[b], acc.at[sets[s][1].at[x]], sems[b])

    def _denom_desc(s, x):
        return pltpu.make_async_copy(
            sets[s][2].at[pl.ds(x * C, C)], dacc.at[sets[s][1].at[x]], semd)

    def _scale(s, x, b):
        t_s = sets[s][2]

        def scale(g, _):
            tv = t_s[pl.ds(x * C + g * L, L)]
            for i in range(L):
                t = tv[i]
                e = g * L + i
                for j in range(H // L):
                    sl = pl.ds(j * L, L)
                    rows[b][e, sl] = rows[b][e, sl] * t
            return _
        lax.fori_loop(0, C // L, scale, None, unroll=5)

    def _step(s, x, b, retire_cond, launch_cond):
        _gather_desc(s, x, b).wait()
        _scale(s, x, b)
        pltpu.async_copy(rows[b], acc.at[sets[s][1].at[x]], sems[b], add=True)
        if layer1:
            pltpu.async_copy(sets[s][2].at[pl.ds(x * C, C)],
                             dacc.at[sets[s][1].at[x]], semd, add=True)
        b2 = (b + 2) % 3

        def retire():
            _scatter_desc(s, x - 1, b2).wait()
            if layer1:
                _denom_desc(s, x - 1).wait()

        if retire_cond is True:
            retire()
        else:
            @pl.when(retire_cond)
            def _():
                retire()

        def launch():
            _gather_desc(s, x + 2, b2).start()

        if launch_cond is True:
            launch()
        elif launch_cond is not False:
            @pl.when(launch_cond)
            def _():
                launch()

    def _pipeline(s, m):
        for d in _load_descs(s, m):
            d.wait()

        idx_s, _, t_s, _ = sets[s]

        def prep(i, _):
            sl = pl.ds(i * L, L)
            t_s[sl] = jnp.exp(t_s[sl])
            idx_s[sl] = idx_s[sl] + c * N
            return _
        lax.fori_loop(0, MEGA // L, prep, None, unroll=5)

        @pl.when(m + 1 < NMEGA)
        def _():
            for d in _load_descs(1 - s, m + 1):
                d.start()

        _gather_desc(s, 0, 0).start()
        _gather_desc(s, 1, 1).start()

        def tri(j, _):
            for i in range(3):
                x = j * 3 + i
                _step(s, x, i, x >= 1, x <= MCH - 3)
            return _
        lax.fori_loop(0, (MCH - 1) // 3, tri, None)
        _step(s, MCH - 1, (MCH - 1) % 3, True, False)
        _scatter_desc(s, MCH - 1, (MCH - 1) % 3).wait()
        if layer1:
            _denom_desc(s, MCH - 1).wait()

    def mega(m, _):
        p = m % 2
        @pl.when(p == 0)
        def _():
            _pipeline(0, m)

        @pl.when(p == 1)
        def _():
            _pipeline(1, m)
        return _

    for d in _load_descs(0, 0):
        d.start()
    lax.fori_loop(0, NMEGA, mega, None)

    plsc.subcore_barrier()

    pltpu.sync_copy(b_hbm.at[pl.ds(c * H, H)], rows1.at[0])
    bs = [rows1[0, pl.ds(j * L, L)] for j in range(H // L)]
    limit = jnp.where(sid == NS - 1, 400, 640)

    def readout(k, _):
        r0 = own + k * ZROWS

        @pl.when(k * ZROWS < limit)
        def _():
            pltpu.sync_copy(acc.at[pl.ds(r0, ZROWS)], rows0)
            if layer1:
                pltpu.sync_copy(dacc.at[pl.ds(r0, ZROWS)], t0.at[pl.ds(0, ZROWS)])
                @pl.when(c == 0)
                def _():
                    pltpu.sync_copy(t0.at[pl.ds(0, ZROWS)],
                                    den_hbm.at[pl.ds(r0, ZROWS)])
            else:
                pltpu.sync_copy(den_hbm.at[pl.ds(r0, ZROWS)],
                                t0.at[pl.ds(0, ZROWS)])

            def fix(g, _):
                dv = t0[pl.ds(g * L, L)]
                dinv = 1.0 / (dv + 1e-16)
                for i in range(L):
                    di = dinv[i]
                    r = g * L + i
                    for j in range(H // L):
                        sl = pl.ds(j * L, L)
                        v = rows0[r, sl] * di + bs[j]
                        if layer1:
                            v = jnp.maximum(v, 0.0)
                        rows0[r, sl] = v
                return _
            lax.fori_loop(0, ZROWS // L, fix, None)
            if layer1:
                pltpu.sync_copy(rows0, out_hbm.at[pl.ds(c * N + r0, ZROWS)])
            else:
                pltpu.sync_copy(rows0, out_hbm.at[pl.ds(r0, ZROWS),
                                                  pl.ds(c * H, H)])
        return _

    lax.fori_loop(0, 640 // ZROWS, readout, None)


def _make_sc(layer1):
    mesh = plsc.VectorSubcoreMesh(core_axis_name="c", subcore_axis_name="s",
                                  num_cores=NC, num_subcores=NS)
    if layer1:
        out_type = (jax.ShapeDtypeStruct((NC * N, H), jnp.float32),
                    jax.ShapeDtypeStruct((N,), jnp.float32))
    else:
        out_type = jax.ShapeDtypeStruct((N, D), jnp.float32)
    return pl.kernel(
        functools.partial(_sc_body, layer1),
        out_type=out_type,
        mesh=mesh,
        scratch_types=[
            pltpu.VMEM_SHARED((N, H), jnp.float32),
            pltpu.VMEM_SHARED((N,), jnp.float32),
            pltpu.VMEM((MEGA,), jnp.int32),
            pltpu.VMEM((MEGA,), jnp.int32),
            pltpu.VMEM((MCH, C), jnp.int32),
            pltpu.VMEM((MCH, C), jnp.int32),
            pltpu.VMEM((MEGA,), jnp.float32),
            pltpu.VMEM((MEGA,), jnp.float32),
            pltpu.VMEM((C, H), jnp.float32),
            pltpu.VMEM((C, H), jnp.float32),
            pltpu.VMEM((C, H), jnp.float32),
            pltpu.SemaphoreType.DMA,
            pltpu.SemaphoreType.DMA,
            pltpu.SemaphoreType.DMA,
            pltpu.SemaphoreType.DMA,
            pltpu.SemaphoreType.DMA,
            pltpu.SemaphoreType.DMA,
            pltpu.SemaphoreType.DMA,
            pltpu.SemaphoreType.DMA,
            pltpu.SemaphoreType.DMA,
        ],
        name="gcn_edge_pass" + ("1" if layer1 else "2"),
    )


_sc_pass1 = _make_sc(True)
_sc_pass2 = _make_sc(False)


def _mm1_body(x_ref, w_ref, o_ref):
    o_ref[0] = jnp.dot(x_ref[...], w_ref[...],
                       preferred_element_type=jnp.float32)


def _mm1(x, W1):
    return pl.pallas_call(
        _mm1_body,
        grid=(N // BN, NC),
        in_specs=[
            pl.BlockSpec((BN, D), lambda i, c: (i, 0)),
            pl.BlockSpec((D, H), lambda i, c: (0, c)),
        ],
        out_specs=pl.BlockSpec((1, BN, H), lambda i, c: (c, i, 0)),
        out_shape=jax.ShapeDtypeStruct((NC, N, H), jnp.float32),
    )(x, W1)


def _mm2_body(s_ref, w_ref, o_ref):
    o_ref[0] = (jnp.dot(s_ref[0], w_ref[:H, :],
                        preferred_element_type=jnp.float32)
                + jnp.dot(s_ref[1], w_ref[H:, :],
                          preferred_element_type=jnp.float32))


def _mm2(s, W2):
    return pl.pallas_call(
        _mm2_body,
        grid=(N // BN, NC),
        in_specs=[
            pl.BlockSpec((NC, BN, H), lambda i, c: (0, i, 0)),
            pl.BlockSpec((D, H), lambda i, c: (0, c)),
        ],
        out_specs=pl.BlockSpec((1, BN, H), lambda i, c: (c, i, 0)),
        out_shape=jax.ShapeDtypeStruct((NC, N, H), jnp.float32),
    )(s, W2)


def kernel(x, edge_index, edge_weight_logits, W1, b1, W2, b2):
    src = edge_index[0]
    dst = edge_index[1].reshape(NS * NMEGA, MCH, C)

    h1 = _mm1(x, W1).reshape(NC * N, H)
    o1, den = _sc_pass1(h1, src, dst, edge_weight_logits, b1)
    h2 = _mm2(o1.reshape(NC, N, H), W2).reshape(NC * N, H)
    out = _sc_pass2(h2, src, dst, edge_weight_logits, b2, den)
    return out[None]


# speedup vs baseline: 1.4942x; 1.4942x over previous
import functools

import jax
import jax.numpy as jnp
from jax import lax
from jax.experimental import pallas as pl
from jax.experimental.pallas import tpu as pltpu
from jax.experimental.pallas import tpu_sc as plsc

N = 10000
E = 160000
D = 256
H = 128
NC = 2
NS = 16
L = 16
EPT = E // NS
C = 80
MEGA = 2000
NMEGA = EPT // MEGA
MCH = MEGA // C
NBUF = 3
NCHUNK = EPT // C
ZROWS = 80
BN = 2000


def _own_range(sid):
    return sid * 640


def _sc_body(layer1, h_hbm, src_hbm, dst2_hbm, logit_hbm, b_hbm, *rest):
    if layer1:
        out_hbm, den_hbm = rest[0], rest[1]
    else:
        den_hbm, out_hbm = rest[0], rest[1]
    (acc, dacc, idx0, idx1, dst0, dst1, t0, t1,
     rows0, rows1, rows2, sem_ld0, sem_ld1, semd,
     semg0, semg1, semg2, sems0, sems1, sems2) = rest[2:]
    c = lax.axis_index("c")
    sid = lax.axis_index("s")
    rows = (rows0, rows1, rows2)
    semg = (semg0, semg1, semg2)
    sems = (sems0, sems1, sems2)
    sets = ((idx0, dst0, t0, sem_ld0), (idx1, dst1, t1, sem_ld1))

    def zero_zb(i, _):
        for j in range(H // L):
            rows0[i, pl.ds(j * L, L)] = jnp.zeros((L,), jnp.float32)
        return _
    lax.fori_loop(0, ZROWS, zero_zb, None)
    for j in range((640 // L) if layer1 else 0):
        t0[pl.ds(j * L, L)] = jnp.zeros((L,), jnp.float32)

    own = _own_range(sid)

    @pl.when(sid < NS - 1)
    def _():
        for k in range(640 // ZROWS):
            pltpu.sync_copy(rows0, acc.at[pl.ds(own + k * ZROWS, ZROWS)])
        if layer1:
            pltpu.sync_copy(t0.at[pl.ds(0, 640)], dacc.at[pl.ds(own, 640)])

    @pl.when(sid == NS - 1)
    def _():
        for k in range(400 // ZROWS):
            pltpu.sync_copy(rows0, acc.at[pl.ds(own + k * ZROWS, ZROWS)])
        if layer1:
            pltpu.sync_copy(t0.at[pl.ds(0, 400)], dacc.at[pl.ds(own, 400)])

    plsc.subcore_barrier()

    def _load_descs(s, m):
        idx_s, dst_s, t_s, sem_s = sets[s]
        base = pl.multiple_of(sid * EPT + m * MEGA, 8)
        blk = sid * NMEGA + m
        return (pltpu.make_async_copy(src_hbm.at[pl.ds(base, MEGA)], idx_s, sem_s),
                pltpu.make_async_copy(logit_hbm.at[pl.ds(base, MEGA)], t_s, sem_s),
                pltpu.make_async_copy(dst2_hbm.at[blk], dst_s, sem_s))

    def _gather_desc(s, x, b):
        return pltpu.make_async_copy(
            h_hbm.at[sets[s][0].at[pl.ds(x * C, C)]], rows[b], semg[b])

    def _scatter_desc(s, x, b):
        return pltpu.make_async_copy(rows[b], acc.at[sets[s][1].at[x]], sems[b])

    def _denom_desc(s, x):
        return pltpu.make_async_copy(
            sets[s][2].at[pl.ds(x * C, C)], dacc.at[sets[s][1].at[x]], semd)

    def _scale(s, x, b):
        t_s = sets[s][2]

        def scale(g, _):
            tv = t_s[pl.ds(x * C + g * L, L)]
            for i in range(L):
                t = tv[i]
                e = g * L + i
                for j in range(H // L):
                    sl = pl.ds(j * L, L)
                    rows[b][e, sl] = rows[b][e, sl] * t
            return _
        lax.fori_loop(0, C // L, scale, None)

    def _step(s, x, b, retire_cond, launch_cond):
        _gather_desc(s, x, b).wait()
        _scale(s, x, b)
        pltpu.async_copy(rows[b], acc.at[sets[s][1].at[x]], sems[b], add=True)
        if layer1:
            pltpu.async_copy(sets[s][2].at[pl.ds(x * C, C)],
                             dacc.at[sets[s][1].at[x]], semd, add=True)
        b2 = (b + 2) % 3

        def retire():
            _scatter_desc(s, x - 1, b2).wait()
            if layer1:
                _denom_desc(s, x - 1).wait()

        if retire_cond is True:
            retire()
        else:
            @pl.when(retire_cond)
            def _():
                retire()

        def launch():
            _gather_desc(s, x + 2, b2).start()

        if launch_cond is True:
            launch()
        elif launch_cond is not False:
            @pl.when(launch_cond)
            def _():
                launch()

    def _pipeline(s, m):
        for d in _load_descs(s, m):
            d.wait()

        idx_s, _, t_s, _ = sets[s]

        def prep(i, _):
            sl = pl.ds(i * L, L)
            t_s[sl] = jnp.exp(t_s[sl])
            idx_s[sl] = idx_s[sl] + c * N
            return _
        lax.fori_loop(0, MEGA // L, prep, None)

        @pl.when(m + 1 < NMEGA)
        def _():
            for d in _load_descs(1 - s, m + 1):
                d.start()

        _gather_desc(s, 0, 0).start()
        _gather_desc(s, 1, 1).start()

        def tri(j, _):
            for i in range(3):
                x = j * 3 + i
                _step(s, x, i, x >= 1, x <= MCH - 3)
            return _
        lax.fori_loop(0, (MCH - 1) // 3, tri, None)
        _step(s, MCH - 1, (MCH - 1) % 3, True, False)
        _scatter_desc(s, MCH - 1, (MCH - 1) % 3).wait()
        if layer1:
            _denom_desc(s, MCH - 1).wait()

    def mega(m, _):
        p = m % 2
        @pl.when(p == 0)
        def _():
            _pipeline(0, m)

        @pl.when(p == 1)
        def _():
            _pipeline(1, m)
        return _

    for d in _load_descs(0, 0):
        d.start()
    lax.fori_loop(0, NMEGA, mega, None)

    plsc.subcore_barrier()

    pltpu.sync_copy(b_hbm.at[pl.ds(c * H, H)], rows1.at[0])
    bs = [rows1[0, pl.ds(j * L, L)] for j in range(H // L)]
    limit = jnp.where(sid == NS - 1, 400, 640)

    def readout(k, _):
        r0 = own + k * ZROWS

        @pl.when(k * ZROWS < limit)
        def _():
            pltpu.sync_copy(acc.at[pl.ds(r0, ZROWS)], rows0)
            if layer1:
                pltpu.sync_copy(dacc.at[pl.ds(r0, ZROWS)], t0.at[pl.ds(0, ZROWS)])
                @pl.when(c == 0)
                def _():
                    pltpu.sync_copy(t0.at[pl.ds(0, ZROWS)],
                                    den_hbm.at[pl.ds(r0, ZROWS)])
            else:
                pltpu.sync_copy(den_hbm.at[pl.ds(r0, ZROWS)],
                                t0.at[pl.ds(0, ZROWS)])

            def fix(g, _):
                dv = t0[pl.ds(g * L, L)]
                dinv = 1.0 / (dv + 1e-16)
                for i in range(L):
                    di = dinv[i]
                    r = g * L + i
                    for j in range(H // L):
                        sl = pl.ds(j * L, L)
                        v = rows0[r, sl] * di + bs[j]
                        if layer1:
                            v = jnp.maximum(v, 0.0)
                        rows0[r, sl] = v
                return _
            lax.fori_loop(0, ZROWS // L, fix, None)
            if layer1:
                pltpu.sync_copy(rows0, out_hbm.at[pl.ds(c * N + r0, ZROWS)])
            else:
                pltpu.sync_copy(rows0, out_hbm.at[pl.ds(r0, ZROWS),
                                                  pl.ds(c * H, H)])
        return _

    lax.fori_loop(0, 640 // ZROWS, readout, None)


def _make_sc(layer1):
    mesh = plsc.VectorSubcoreMesh(core_axis_name="c", subcore_axis_name="s",
                                  num_cores=NC, num_subcores=NS)
    if layer1:
        out_type = (jax.ShapeDtypeStruct((NC * N, H), jnp.float32),
                    jax.ShapeDtypeStruct((N,), jnp.float32))
    else:
        out_type = jax.ShapeDtypeStruct((N, D), jnp.float32)
    return pl.kernel(
        functools.partial(_sc_body, layer1),
        out_type=out_type,
        mesh=mesh,
        scratch_types=[
            pltpu.VMEM_SHARED((N, H), jnp.float32),
            pltpu.VMEM_SHARED((N,), jnp.float32),
            pltpu.VMEM((MEGA,), jnp.int32),
            pltpu.VMEM((MEGA,), jnp.int32),
            pltpu.VMEM((MCH, C), jnp.int32),
            pltpu.VMEM((MCH, C), jnp.int32),
            pltpu.VMEM((MEGA,), jnp.float32),
            pltpu.VMEM((MEGA,), jnp.float32),
            pltpu.VMEM((C, H), jnp.float32),
            pltpu.VMEM((C, H), jnp.float32),
            pltpu.VMEM((C, H), jnp.float32),
            pltpu.SemaphoreType.DMA,
            pltpu.SemaphoreType.DMA,
            pltpu.SemaphoreType.DMA,
            pltpu.SemaphoreType.DMA,
            pltpu.SemaphoreType.DMA,
            pltpu.SemaphoreType.DMA,
            pltpu.SemaphoreType.DMA,
            pltpu.SemaphoreType.DMA,
            pltpu.SemaphoreType.DMA,
        ],
        name="gcn_edge_pass" + ("1" if layer1 else "2"),
    )


_sc_pass1 = _make_sc(True)
_sc_pass2 = _make_sc(False)


def _mm1_body(x_ref, w_ref, o_ref):
    o_ref[0] = jnp.dot(x_ref[...], w_ref[...],
                       preferred_element_type=jnp.float32)


def _mm1(x, W1):
    return pl.pallas_call(
        _mm1_body,
        grid=(N // BN, NC),
        in_specs=[
            pl.BlockSpec((BN, D), lambda i, c: (i, 0)),
            pl.BlockSpec((D, H), lambda i, c: (0, c)),
        ],
        out_specs=pl.BlockSpec((1, BN, H), lambda i, c: (c, i, 0)),
        out_shape=jax.ShapeDtypeStruct((NC, N, H), jnp.float32),
    )(x, W1)


def _mm2_body(s_ref, w_ref, o_ref):
    o_ref[0] = (jnp.dot(s_ref[0], w_ref[:H, :],
                        preferred_element_type=jnp.float32)
                + jnp.dot(s_ref[1], w_ref[H:, :],
                          preferred_element_type=jnp.float32))


def _mm2(s, W2):
    return pl.pallas_call(
        _mm2_body,
        grid=(N // BN, NC),
        in_specs=[
            pl.BlockSpec((NC, BN, H), lambda i, c: (0, i, 0)),
            pl.BlockSpec((D, H), lambda i, c: (0, c)),
        ],
        out_specs=pl.BlockSpec((1, BN, H), lambda i, c: (c, i, 0)),
        out_shape=jax.ShapeDtypeStruct((NC, N, H), jnp.float32),
    )(s, W2)


def kernel(x, edge_index, edge_weight_logits, W1, b1, W2, b2):
    src = edge_index[0]
    dst = edge_index[1].reshape(NS * NMEGA, MCH, C)

    h1 = _mm1(x, W1).reshape(NC * N, H)
    o1, den = _sc_pass1(h1, src, dst, edge_weight_logits, b1)
    h2 = _mm2(o1.reshape(NC, N, H), W2).reshape(NC * N, H)
    out = _sc_pass2(h2, src, dst, edge_weight_logits, b2, den)
    return out[None]
